# baseline (speedup 1.0000x reference)
import functools

import jax
import jax.numpy as jnp
from jax import lax
from jax.experimental import pallas as pl
from jax.experimental.pallas import tpu as pltpu
from jax.experimental.pallas import tpu_sc as plsc

C = 1000
CP = 1024
D = 256
B = 16384
BLK = 2048
NB = B // BLK

NH = 4
NG = 8
W = D // NG
RPT = B // NH
RCH = 512
NCHT = RPT // RCH
GRP = 16


def _offsets_body(pred_ref, truth_ref, off_ref):
    pred = pred_ref[...]
    truth = truth_ref[...]
    pn = jnp.sqrt(jnp.sum(pred * pred, axis=1, keepdims=True))
    p_ = pred / (pn + 1e-10)
    tn = jnp.sqrt(jnp.sum(truth * truth, axis=1, keepdims=True))
    t_ = truth / (tn + 1e-10)
    off_ref[...] = (p_ - t_) ** 2


def _sc_segsum(off_hbm, lab_hbm, out_sum, out_cnt,
               acc_v, cnt_v, buf_v, lab_v, sem_in, sem_lab):
    c = lax.axis_index("c")
    s = lax.axis_index("s")
    wid = c * 16 + s
    g = wid % NG
    h = wid // NG
    row0 = h * RPT
    col0 = g * W

    zeros16 = jnp.zeros((16,), jnp.float32)
    ones16 = jnp.ones((16,), jnp.float32)
    iota16 = lax.iota(jnp.int32, 16)
    col_z = jnp.broadcast_to(jnp.int32(0), (16,))

    @plsc.parallel_loop(0, CP, GRP)
    def zrow(r):
        for rr in range(GRP):
            for jb in range(W // 16):
                acc_v[r + rr, pl.ds(jb * 16, 16)] = zeros16
            cnt_v[r + rr, :] = zeros16

    cps = [None, None]
    lps = [None, None]
    cps[0] = pltpu.async_copy(
        off_hbm.at[pl.ds(row0, RCH), pl.ds(col0, W)], buf_v.at[0], sem_in)
    lps[0] = pltpu.async_copy(
        lab_hbm.at[pl.ds(row0, RCH)], lab_v.at[0], sem_lab)

    for ch in range(NCHT):
        b = ch % 2
        if ch + 1 < NCHT:
            nb = (ch + 1) % 2
            cps[nb] = pltpu.async_copy(
                off_hbm.at[pl.ds(row0 + (ch + 1) * RCH, RCH), pl.ds(col0, W)],
                buf_v.at[nb], sem_in)
            lps[nb] = pltpu.async_copy(
                lab_hbm.at[pl.ds(row0 + (ch + 1) * RCH, RCH)],
                lab_v.at[nb], sem_lab)
        cps[b].wait()
        lps[b].wait()

        @plsc.parallel_loop(0, RCH // GRP, 1, unroll=2)
        def grp_body(gi):
            labs = lab_v[b, pl.ds(gi * GRP, GRP)]
            rows = jnp.broadcast_to(gi * GRP, (16,)) + iota16
            for jb in range(W // 16):
                cols = [jnp.broadcast_to(jnp.int32(jb * 16 + j), (16,))
                        for j in range(16)]
                vals = [plsc.load_gather(buf_v.at[b], [rows, cols[j]])
                        for j in range(16)]
                for j in range(16):
                    plsc.addupdate_scatter(acc_v, [labs, cols[j]], vals[j])

            @pl.when(g == 0)
            def _cnt():
                plsc.addupdate_scatter(cnt_v, [labs, col_z], ones16)

    pltpu.sync_copy(acc_v, out_sum.at[h, g])

    @pl.when(g == 0)
    def _cnt_out():
        pltpu.sync_copy(cnt_v, out_cnt.at[h])


_sc_segsum_call = functools.partial(
    pl.kernel,
    out_type=[jax.ShapeDtypeStruct((NH, NG, CP, W), jnp.float32),
              jax.ShapeDtypeStruct((NH, CP, 16), jnp.float32)],
    mesh=plsc.VectorSubcoreMesh(core_axis_name="c", subcore_axis_name="s"),
    compiler_params=pltpu.CompilerParams(use_tc_tiling_on_sc=False,
                                         needs_layout_passes=False),
    scratch_types=[
        pltpu.VMEM((CP, W), jnp.float32),
        pltpu.VMEM((CP, 16), jnp.float32),
        pltpu.VMEM((2, RCH, W), jnp.float32),
        pltpu.VMEM((2, RCH), jnp.int32),
        pltpu.SemaphoreType.DMA,
        pltpu.SemaphoreType.DMA,
    ],
)(_sc_segsum)


def _weights_body(part_ref, cnt_ref, mean_ref, w_ref):
    pieces = []
    for gidx in range(NG):
        acc = part_ref[0, gidx]
        for hidx in range(1, NH):
            acc = acc + part_ref[hidx, gidx]
        pieces.append(acc)
    sums = jnp.concatenate(pieces, axis=1)
    cnt = (cnt_ref[0, :, 0:1] + cnt_ref[1, :, 0:1]
           + cnt_ref[2, :, 0:1] + cnt_ref[3, :, 0:1])
    mean = sums / jnp.maximum(cnt, 1.0)
    mask = mean > 0.0
    big = jnp.where(mask, mean, jnp.inf)
    col_min = jnp.min(big, axis=1, keepdims=True)
    row_min = jnp.min(big, axis=0, keepdims=True)
    col_min = jnp.where(col_min < jnp.inf, col_min, 1.0)
    row_min = jnp.where(row_min < jnp.inf, row_min, 1.0)
    safe = jnp.where(mask, mean, 1.0)
    w1 = jnp.log(safe / row_min) + 1.0
    w2 = jnp.log(safe / col_min) + 1.0
    w = jnp.where(mask, w1 * w2, 1.0)
    mean_ref[...] = mean
    w_ref[...] = w


@jax.jit
def kernel(batch_pred, batch_truth, batch_label):
    offsets = pl.pallas_call(
        _offsets_body,
        grid=(NB,),
        in_specs=[
            pl.BlockSpec((BLK, D), lambda i: (i, 0)),
            pl.BlockSpec((BLK, D), lambda i: (i, 0)),
        ],
        out_specs=pl.BlockSpec((BLK, D), lambda i: (i, 0)),
        out_shape=jax.ShapeDtypeStruct((B, D), jnp.float32),
    )(batch_pred, batch_truth)

    parts, cnts = _sc_segsum_call(offsets, batch_label)

    mean_p, w_p = pl.pallas_call(
        _weights_body,
        out_shape=[
            jax.ShapeDtypeStruct((CP, D), jnp.float32),
            jax.ShapeDtypeStruct((CP, D), jnp.float32),
        ],
    )(parts, cnts)
    return (mean_p[:C], w_p[:C])


# speedup vs baseline: 1.9637x; 1.9637x over previous
import functools

import jax
import jax.numpy as jnp
from jax import lax
from jax.experimental import pallas as pl
from jax.experimental.pallas import tpu as pltpu
from jax.experimental.pallas import tpu_sc as plsc

C = 1000
CP = 1024
D = 256
B = 16384
BLK = 2048
NB = B // BLK

NH = 4
NG = 8
W = D // NG
RPT = B // NH
RCH = 512
NCHT = RPT // RCH
GRP = 16


def _offsets_body(pred_ref, truth_ref, off_ref):
    pred = pred_ref[...]
    truth = truth_ref[...]
    pn = jnp.sqrt(jnp.sum(pred * pred, axis=1, keepdims=True))
    p_ = pred / (pn + 1e-10)
    tn = jnp.sqrt(jnp.sum(truth * truth, axis=1, keepdims=True))
    t_ = truth / (tn + 1e-10)
    off_ref[...] = (p_ - t_) ** 2


def _sc_segsum(off_hbm, lab_hbm, out_sum, out_cnt,
               acc_v, cnt_v, buf_v, lab_v, sem_in, sem_lab):
    c = lax.axis_index("c")
    s = lax.axis_index("s")
    wid = c * 16 + s
    g = wid % NG
    h = wid // NG
    row0 = h * RPT
    col0 = g * W

    zeros16 = jnp.zeros((16,), jnp.float32)
    ones16 = jnp.ones((16,), jnp.float32)
    iota16 = lax.iota(jnp.int32, 16)
    col_z = jnp.broadcast_to(jnp.int32(0), (16,))

    @plsc.parallel_loop(0, CP, GRP)
    def zrow(r):
        for rr in range(GRP):
            for jb in range(W // 16):
                acc_v[r + rr, pl.ds(jb * 16, 16)] = zeros16
            cnt_v[r + rr, :] = zeros16

    cps = [None, None]
    lps = [None, None]
    cps[0] = pltpu.async_copy(
        off_hbm.at[pl.ds(row0, RCH), pl.ds(col0, W)], buf_v.at[0], sem_in)
    lps[0] = pltpu.async_copy(
        lab_hbm.at[pl.ds(row0, RCH)], lab_v.at[0], sem_lab)

    for ch in range(NCHT):
        b = ch % 2
        if ch + 1 < NCHT:
            nb = (ch + 1) % 2
            cps[nb] = pltpu.async_copy(
                off_hbm.at[pl.ds(row0 + (ch + 1) * RCH, RCH), pl.ds(col0, W)],
                buf_v.at[nb], sem_in)
            lps[nb] = pltpu.async_copy(
                lab_hbm.at[pl.ds(row0 + (ch + 1) * RCH, RCH)],
                lab_v.at[nb], sem_lab)
        cps[b].wait()
        lps[b].wait()

        cols = [lax.iota(jnp.int32, 16) + jb * 16 for jb in range(W // 16)]

        @plsc.parallel_loop(0, RCH // GRP, 1, unroll=2)
        def grp_body(gi):
            base_r = gi * GRP
            labs1 = [plsc.load_gather(
                lab_v.at[b], [jnp.broadcast_to(base_r + j, (16,))])
                for j in range(GRP)]
            for j in range(GRP):
                for jb in range(W // 16):
                    vals = buf_v[b, base_r + j, pl.ds(jb * 16, 16)]
                    plsc.addupdate_scatter(acc_v, [labs1[j], cols[jb]], vals)

            @pl.when(g == 0)
            def _cnt():
                labs = lab_v[b, pl.ds(base_r, GRP)]
                plsc.addupdate_scatter(cnt_v, [labs, col_z], ones16)

    pltpu.sync_copy(acc_v, out_sum.at[h, g])

    @pl.when(g == 0)
    def _cnt_out():
        pltpu.sync_copy(cnt_v, out_cnt.at[h])


_sc_segsum_call = functools.partial(
    pl.kernel,
    out_type=[jax.ShapeDtypeStruct((NH, NG, CP, W), jnp.float32),
              jax.ShapeDtypeStruct((NH, CP, 16), jnp.float32)],
    mesh=plsc.VectorSubcoreMesh(core_axis_name="c", subcore_axis_name="s"),
    compiler_params=pltpu.CompilerParams(use_tc_tiling_on_sc=False,
                                         needs_layout_passes=False),
    scratch_types=[
        pltpu.VMEM((CP, W), jnp.float32),
        pltpu.VMEM((CP, 16), jnp.float32),
        pltpu.VMEM((2, RCH, W), jnp.float32),
        pltpu.VMEM((2, RCH), jnp.int32),
        pltpu.SemaphoreType.DMA,
        pltpu.SemaphoreType.DMA,
    ],
)(_sc_segsum)


def _weights_body(part_ref, cnt_ref, mean_ref, w_ref):
    pieces = []
    for gidx in range(NG):
        acc = part_ref[0, gidx]
        for hidx in range(1, NH):
            acc = acc + part_ref[hidx, gidx]
        pieces.append(acc)
    sums = jnp.concatenate(pieces, axis=1)
    cnt = (cnt_ref[0, :, 0:1] + cnt_ref[1, :, 0:1]
           + cnt_ref[2, :, 0:1] + cnt_ref[3, :, 0:1])
    mean = sums / jnp.maximum(cnt, 1.0)
    mask = mean > 0.0
    big = jnp.where(mask, mean, jnp.inf)
    col_min = jnp.min(big, axis=1, keepdims=True)
    row_min = jnp.min(big, axis=0, keepdims=True)
    col_min = jnp.where(col_min < jnp.inf, col_min, 1.0)
    row_min = jnp.where(row_min < jnp.inf, row_min, 1.0)
    safe = jnp.where(mask, mean, 1.0)
    w1 = jnp.log(safe / row_min) + 1.0
    w2 = jnp.log(safe / col_min) + 1.0
    w = jnp.where(mask, w1 * w2, 1.0)
    mean_ref[...] = mean
    w_ref[...] = w


@jax.jit
def kernel(batch_pred, batch_truth, batch_label):
    offsets = pl.pallas_call(
        _offsets_body,
        grid=(NB,),
        in_specs=[
            pl.BlockSpec((BLK, D), lambda i: (i, 0)),
            pl.BlockSpec((BLK, D), lambda i: (i, 0)),
        ],
        out_specs=pl.BlockSpec((BLK, D), lambda i: (i, 0)),
        out_shape=jax.ShapeDtypeStruct((B, D), jnp.float32),
    )(batch_pred, batch_truth)

    parts, cnts = _sc_segsum_call(offsets, batch_label)

    mean_p, w_p = pl.pallas_call(
        _weights_body,
        out_shape=[
            jax.ShapeDtypeStruct((CP, D), jnp.float32),
            jax.ShapeDtypeStruct((CP, D), jnp.float32),
        ],
    )(parts, cnts)
    return (mean_p[:C], w_p[:C])


# speedup vs baseline: 2.1231x; 1.0812x over previous
import functools

import jax
import jax.numpy as jnp
from jax import lax
from jax.experimental import pallas as pl
from jax.experimental.pallas import tpu as pltpu
from jax.experimental.pallas import tpu_sc as plsc

C = 1000
CP = 1024
D = 256
B = 16384
BLK = 2048
NB = B // BLK

NH = 4
NG = 8
W = D // NG
RPT = B // NH
RCH = 512
NCHT = RPT // RCH
GRP = 16


def _offsets_body(pred_ref, truth_ref, off_ref):
    pred = pred_ref[...]
    truth = truth_ref[...]
    pn = jnp.sqrt(jnp.sum(pred * pred, axis=1, keepdims=True))
    p_ = pred / (pn + 1e-10)
    tn = jnp.sqrt(jnp.sum(truth * truth, axis=1, keepdims=True))
    t_ = truth / (tn + 1e-10)
    off_ref[...] = (p_ - t_) ** 2


def _sc_segsum(off_hbm, lab_hbm, out_sum, out_cnt,
               acc_v, cnt_v, buf_v, lab_v, sem_in, sem_lab):
    c = lax.axis_index("c")
    s = lax.axis_index("s")
    wid = c * 16 + s
    g = wid % NG
    h = wid // NG
    row0 = h * RPT
    col0 = g * W

    zeros16 = jnp.zeros((16,), jnp.float32)
    ones16 = jnp.ones((16,), jnp.float32)
    iota16 = lax.iota(jnp.int32, 16)
    col_z = jnp.broadcast_to(jnp.int32(0), (16,))

    @plsc.parallel_loop(0, CP, GRP)
    def zrow(r):
        for rr in range(GRP):
            for jb in range(W // 16):
                acc_v[r + rr, pl.ds(jb * 16, 16)] = zeros16
            cnt_v[r + rr, :] = zeros16

    cps = [None, None]
    lps = [None, None]
    cps[0] = pltpu.async_copy(
        off_hbm.at[pl.ds(row0, RCH), pl.ds(col0, W)], buf_v.at[0], sem_in)
    lps[0] = pltpu.async_copy(
        lab_hbm.at[pl.ds(row0, RCH)], lab_v.at[0], sem_lab)

    for ch in range(NCHT):
        b = ch % 2
        if ch + 1 < NCHT:
            nb = (ch + 1) % 2
            cps[nb] = pltpu.async_copy(
                off_hbm.at[pl.ds(row0 + (ch + 1) * RCH, RCH), pl.ds(col0, W)],
                buf_v.at[nb], sem_in)
            lps[nb] = pltpu.async_copy(
                lab_hbm.at[pl.ds(row0 + (ch + 1) * RCH, RCH)],
                lab_v.at[nb], sem_lab)
        cps[b].wait()
        lps[b].wait()

        cols = [lax.iota(jnp.int32, 16) + jb * 16 for jb in range(W // 16)]

        @plsc.parallel_loop(0, RCH // GRP, 1, unroll=4)
        def grp_body(gi):
            base_r = gi * GRP
            labs1 = [plsc.load_gather(
                lab_v.at[b], [jnp.broadcast_to(base_r + j, (16,))])
                for j in range(GRP)]
            for j in range(GRP):
                for jb in range(W // 16):
                    vals = buf_v[b, base_r + j, pl.ds(jb * 16, 16)]
                    plsc.addupdate_scatter(acc_v, [labs1[j], cols[jb]], vals)

            @pl.when(g == 0)
            def _cnt():
                labs = lab_v[b, pl.ds(base_r, GRP)]
                plsc.addupdate_scatter(cnt_v, [labs, col_z], ones16)

    pltpu.sync_copy(acc_v, out_sum.at[h, g])

    @pl.when(g == 0)
    def _cnt_out():
        pltpu.sync_copy(cnt_v, out_cnt.at[h])


_sc_segsum_call = functools.partial(
    pl.kernel,
    out_type=[jax.ShapeDtypeStruct((NH, NG, CP, W), jnp.float32),
              jax.ShapeDtypeStruct((NH, CP, 16), jnp.float32)],
    mesh=plsc.VectorSubcoreMesh(core_axis_name="c", subcore_axis_name="s"),
    compiler_params=pltpu.CompilerParams(use_tc_tiling_on_sc=False,
                                         needs_layout_passes=False),
    scratch_types=[
        pltpu.VMEM((CP, W), jnp.float32),
        pltpu.VMEM((CP, 16), jnp.float32),
        pltpu.VMEM((2, RCH, W), jnp.float32),
        pltpu.VMEM((2, RCH), jnp.int32),
        pltpu.SemaphoreType.DMA,
        pltpu.SemaphoreType.DMA,
    ],
)(_sc_segsum)


def _weights_body(part_ref, cnt_ref, mean_ref, w_ref):
    pieces = []
    for gidx in range(NG):
        acc = part_ref[0, gidx]
        for hidx in range(1, NH):
            acc = acc + part_ref[hidx, gidx]
        pieces.append(acc)
    sums = jnp.concatenate(pieces, axis=1)
    cnt = (cnt_ref[0, :, 0:1] + cnt_ref[1, :, 0:1]
           + cnt_ref[2, :, 0:1] + cnt_ref[3, :, 0:1])
    mean = sums / jnp.maximum(cnt, 1.0)
    mask = mean > 0.0
    big = jnp.where(mask, mean, jnp.inf)
    col_min = jnp.min(big, axis=1, keepdims=True)
    row_min = jnp.min(big, axis=0, keepdims=True)
    col_min = jnp.where(col_min < jnp.inf, col_min, 1.0)
    row_min = jnp.where(row_min < jnp.inf, row_min, 1.0)
    safe = jnp.where(mask, mean, 1.0)
    w1 = jnp.log(safe / row_min) + 1.0
    w2 = jnp.log(safe / col_min) + 1.0
    w = jnp.where(mask, w1 * w2, 1.0)
    mean_ref[...] = mean[:C]
    w_ref[...] = w[:C]


@jax.jit
def kernel(batch_pred, batch_truth, batch_label):
    offsets = pl.pallas_call(
        _offsets_body,
        grid=(NB,),
        in_specs=[
            pl.BlockSpec((BLK, D), lambda i: (i, 0)),
            pl.BlockSpec((BLK, D), lambda i: (i, 0)),
        ],
        out_specs=pl.BlockSpec((BLK, D), lambda i: (i, 0)),
        out_shape=jax.ShapeDtypeStruct((B, D), jnp.float32),
    )(batch_pred, batch_truth)

    parts, cnts = _sc_segsum_call(offsets, batch_label)

    mean_p, w_p = pl.pallas_call(
        _weights_body,
        out_shape=[
            jax.ShapeDtypeStruct((C, D), jnp.float32),
            jax.ShapeDtypeStruct((C, D), jnp.float32),
        ],
    )(parts, cnts)
    return (mean_p, w_p)


# speedup vs baseline: 2.1812x; 1.0274x over previous
import functools

import jax
import jax.numpy as jnp
from jax import lax
from jax.experimental import pallas as pl
from jax.experimental.pallas import tpu as pltpu
from jax.experimental.pallas import tpu_sc as plsc

C = 1000
CP = 1024
D = 256
B = 16384
BLK = 2048
NB = B // BLK

NH = 4
NG = 8
W = D // NG
RPT = B // NH
RCH = 1024
NCHT = RPT // RCH
GRP = 16


def _offsets_body(pred_ref, truth_ref, off_ref):
    pred = pred_ref[...]
    truth = truth_ref[...]
    pn = jnp.sqrt(jnp.sum(pred * pred, axis=1, keepdims=True))
    p_ = pred / (pn + 1e-10)
    tn = jnp.sqrt(jnp.sum(truth * truth, axis=1, keepdims=True))
    t_ = truth / (tn + 1e-10)
    off_ref[...] = (p_ - t_) ** 2


def _sc_segsum(off_hbm, lab_hbm, out_sum, out_cnt,
               acc_v, cnt_v, buf_v, lab_v, sem_in, sem_lab):
    c = lax.axis_index("c")
    s = lax.axis_index("s")
    wid = c * 16 + s
    g = wid % NG
    h = wid // NG
    row0 = h * RPT
    col0 = g * W

    zeros16 = jnp.zeros((16,), jnp.float32)
    ones16 = jnp.ones((16,), jnp.float32)
    iota16 = lax.iota(jnp.int32, 16)
    col_z = jnp.broadcast_to(jnp.int32(0), (16,))

    @plsc.parallel_loop(0, CP, GRP)
    def zrow(r):
        for rr in range(GRP):
            for jb in range(W // 16):
                acc_v[r + rr, pl.ds(jb * 16, 16)] = zeros16
            cnt_v[r + rr, :] = zeros16

    cps = [None, None]
    lps = [None, None]
    cps[0] = pltpu.async_copy(
        off_hbm.at[pl.ds(row0, RCH), pl.ds(col0, W)], buf_v.at[0], sem_in)
    lps[0] = pltpu.async_copy(
        lab_hbm.at[pl.ds(row0, RCH)], lab_v.at[0], sem_lab)

    for ch in range(NCHT):
        b = ch % 2
        if ch + 1 < NCHT:
            nb = (ch + 1) % 2
            cps[nb] = pltpu.async_copy(
                off_hbm.at[pl.ds(row0 + (ch + 1) * RCH, RCH), pl.ds(col0, W)],
                buf_v.at[nb], sem_in)
            lps[nb] = pltpu.async_copy(
                lab_hbm.at[pl.ds(row0 + (ch + 1) * RCH, RCH)],
                lab_v.at[nb], sem_lab)
        cps[b].wait()
        lps[b].wait()

        cols = [lax.iota(jnp.int32, 16) + jb * 16 for jb in range(W // 16)]

        @plsc.parallel_loop(0, RCH // GRP, 1, unroll=4)
        def grp_body(gi):
            base_r = gi * GRP
            labs1 = [plsc.load_gather(
                lab_v.at[b], [jnp.broadcast_to(base_r + j, (16,))])
                for j in range(GRP)]
            for j in range(GRP):
                for jb in range(W // 16):
                    vals = buf_v[b, base_r + j, pl.ds(jb * 16, 16)]
                    plsc.addupdate_scatter(acc_v, [labs1[j], cols[jb]], vals)

            @pl.when(g == 0)
            def _cnt():
                labs = lab_v[b, pl.ds(base_r, GRP)]
                plsc.addupdate_scatter(cnt_v, [labs, col_z], ones16)

    pltpu.sync_copy(acc_v, out_sum.at[h, g])

    @pl.when(g == 0)
    def _cnt_out():
        pltpu.sync_copy(cnt_v, out_cnt.at[h])


_sc_segsum_call = functools.partial(
    pl.kernel,
    out_type=[jax.ShapeDtypeStruct((NH, NG, CP, W), jnp.float32),
              jax.ShapeDtypeStruct((NH, CP, 16), jnp.float32)],
    mesh=plsc.VectorSubcoreMesh(core_axis_name="c", subcore_axis_name="s"),
    compiler_params=pltpu.CompilerParams(use_tc_tiling_on_sc=False,
                                         needs_layout_passes=False),
    scratch_types=[
        pltpu.VMEM((CP, W), jnp.float32),
        pltpu.VMEM((CP, 16), jnp.float32),
        pltpu.VMEM((2, RCH, W), jnp.float32),
        pltpu.VMEM((2, RCH), jnp.int32),
        pltpu.SemaphoreType.DMA,
        pltpu.SemaphoreType.DMA,
    ],
)(_sc_segsum)


def _weights_body(part_ref, cnt_ref, mean_ref, w_ref):
    pieces = []
    for gidx in range(NG):
        acc = part_ref[0, gidx]
        for hidx in range(1, NH):
            acc = acc + part_ref[hidx, gidx]
        pieces.append(acc)
    sums = jnp.concatenate(pieces, axis=1)
    cnt = (cnt_ref[0, :, 0:1] + cnt_ref[1, :, 0:1]
           + cnt_ref[2, :, 0:1] + cnt_ref[3, :, 0:1])
    mean = sums / jnp.maximum(cnt, 1.0)
    mask = mean > 0.0
    big = jnp.where(mask, mean, jnp.inf)
    col_min = jnp.min(big, axis=1, keepdims=True)
    row_min = jnp.min(big, axis=0, keepdims=True)
    col_min = jnp.where(col_min < jnp.inf, col_min, 1.0)
    row_min = jnp.where(row_min < jnp.inf, row_min, 1.0)
    safe = jnp.where(mask, mean, 1.0)
    w1 = jnp.log(safe / row_min) + 1.0
    w2 = jnp.log(safe / col_min) + 1.0
    w = jnp.where(mask, w1 * w2, 1.0)
    mean_ref[...] = mean[:C]
    w_ref[...] = w[:C]


@jax.jit
def kernel(batch_pred, batch_truth, batch_label):
    offsets = pl.pallas_call(
        _offsets_body,
        grid=(NB,),
        in_specs=[
            pl.BlockSpec((BLK, D), lambda i: (i, 0)),
            pl.BlockSpec((BLK, D), lambda i: (i, 0)),
        ],
        out_specs=pl.BlockSpec((BLK, D), lambda i: (i, 0)),
        out_shape=jax.ShapeDtypeStruct((B, D), jnp.float32),
    )(batch_pred, batch_truth)

    parts, cnts = _sc_segsum_call(offsets, batch_label)

    mean_p, w_p = pl.pallas_call(
        _weights_body,
        out_shape=[
            jax.ShapeDtypeStruct((C, D), jnp.float32),
            jax.ShapeDtypeStruct((C, D), jnp.float32),
        ],
    )(parts, cnts)
    return (mean_p, w_p)


# speedup vs baseline: 2.2204x; 1.0180x over previous
import functools

import jax
import jax.numpy as jnp
from jax import lax
from jax.experimental import pallas as pl
from jax.experimental.pallas import tpu as pltpu
from jax.experimental.pallas import tpu_sc as plsc

C = 1000
CP = 1024
D = 256
B = 16384
BLK = 2048
NB = B // BLK

NH = 4
NG = 8
W = D // NG
RPT = B // NH
RCH = 1024
NCHT = RPT // RCH
GRP = 16


def _offsets_body(pred_ref, truth_ref, off_ref):
    pred = pred_ref[...]
    truth = truth_ref[...]
    pn = jnp.sqrt(jnp.sum(pred * pred, axis=1, keepdims=True))
    p_ = pred / (pn + 1e-10)
    tn = jnp.sqrt(jnp.sum(truth * truth, axis=1, keepdims=True))
    t_ = truth / (tn + 1e-10)
    off_ref[...] = (p_ - t_) ** 2


def _sc_segsum(off_hbm, lab_hbm, out_sum, out_cnt,
               acc_v, cnt_v, buf_v, lab_v, sem_in, sem_lab):
    c = lax.axis_index("c")
    s = lax.axis_index("s")
    wid = c * 16 + s
    g = wid % NG
    h = wid // NG
    row0 = h * RPT
    col0 = g * W

    zeros16 = jnp.zeros((16,), jnp.float32)
    ones16 = jnp.ones((16,), jnp.float32)
    iota16 = lax.iota(jnp.int32, 16)
    col_z = jnp.broadcast_to(jnp.int32(0), (16,))

    cps = [None, None]
    lps = [None, None]
    cps[0] = pltpu.async_copy(
        off_hbm.at[pl.ds(row0, RCH), pl.ds(col0, W)], buf_v.at[0], sem_in)
    lps[0] = pltpu.async_copy(
        lab_hbm.at[pl.ds(row0, RCH)], lab_v.at[0], sem_lab)

    @plsc.parallel_loop(0, CP, GRP)
    def zrow(r):
        for rr in range(GRP):
            for jb in range(W // 16):
                acc_v[r + rr, pl.ds(jb * 16, 16)] = zeros16

    @pl.when(g == 0)
    def _zcnt():
        @plsc.parallel_loop(0, CP, GRP)
        def zcnt(r):
            for rr in range(GRP):
                cnt_v[r + rr, :] = zeros16

    for ch in range(NCHT):
        b = ch % 2
        if ch + 1 < NCHT:
            nb = (ch + 1) % 2
            cps[nb] = pltpu.async_copy(
                off_hbm.at[pl.ds(row0 + (ch + 1) * RCH, RCH), pl.ds(col0, W)],
                buf_v.at[nb], sem_in)
            lps[nb] = pltpu.async_copy(
                lab_hbm.at[pl.ds(row0 + (ch + 1) * RCH, RCH)],
                lab_v.at[nb], sem_lab)
        cps[b].wait()
        lps[b].wait()

        cols = [lax.iota(jnp.int32, 16) + jb * 16 for jb in range(W // 16)]

        @plsc.parallel_loop(0, RCH // GRP, 1, unroll=4)
        def grp_body(gi):
            base_r = gi * GRP
            labs1 = [plsc.load_gather(
                lab_v.at[b], [jnp.broadcast_to(base_r + j, (16,))])
                for j in range(GRP)]
            for j in range(GRP):
                for jb in range(W // 16):
                    vals = buf_v[b, base_r + j, pl.ds(jb * 16, 16)]
                    plsc.addupdate_scatter(acc_v, [labs1[j], cols[jb]], vals)

            @pl.when(g == 0)
            def _cnt():
                labs = lab_v[b, pl.ds(base_r, GRP)]
                plsc.addupdate_scatter(cnt_v, [labs, col_z], ones16)

    pltpu.sync_copy(acc_v, out_sum.at[h, g])

    @pl.when(g == 0)
    def _cnt_out():
        pltpu.sync_copy(cnt_v, out_cnt.at[h])


_sc_segsum_call = functools.partial(
    pl.kernel,
    out_type=[jax.ShapeDtypeStruct((NH, NG, CP, W), jnp.float32),
              jax.ShapeDtypeStruct((NH, CP, 16), jnp.float32)],
    mesh=plsc.VectorSubcoreMesh(core_axis_name="c", subcore_axis_name="s"),
    compiler_params=pltpu.CompilerParams(use_tc_tiling_on_sc=False,
                                         needs_layout_passes=False),
    scratch_types=[
        pltpu.VMEM((CP, W), jnp.float32),
        pltpu.VMEM((CP, 16), jnp.float32),
        pltpu.VMEM((2, RCH, W), jnp.float32),
        pltpu.VMEM((2, RCH), jnp.int32),
        pltpu.SemaphoreType.DMA,
        pltpu.SemaphoreType.DMA,
    ],
)(_sc_segsum)


def _weights_body(part_ref, cnt_ref, mean_ref, w_ref):
    pieces = []
    for gidx in range(NG):
        acc = part_ref[0, gidx]
        for hidx in range(1, NH):
            acc = acc + part_ref[hidx, gidx]
        pieces.append(acc)
    sums = jnp.concatenate(pieces, axis=1)
    cnt = (cnt_ref[0, :, 0:1] + cnt_ref[1, :, 0:1]
           + cnt_ref[2, :, 0:1] + cnt_ref[3, :, 0:1])
    mean = sums / jnp.maximum(cnt, 1.0)
    mask = mean > 0.0
    big = jnp.where(mask, mean, jnp.inf)
    col_min = jnp.min(big, axis=1, keepdims=True)
    row_min = jnp.min(big, axis=0, keepdims=True)
    col_min = jnp.where(col_min < jnp.inf, col_min, 1.0)
    row_min = jnp.where(row_min < jnp.inf, row_min, 1.0)
    safe = jnp.where(mask, mean, 1.0)
    w1 = jnp.log(safe / row_min) + 1.0
    w2 = jnp.log(safe / col_min) + 1.0
    w = jnp.where(mask, w1 * w2, 1.0)
    mean_ref[...] = mean[:C]
    w_ref[...] = w[:C]


@jax.jit
def kernel(batch_pred, batch_truth, batch_label):
    offsets = pl.pallas_call(
        _offsets_body,
        grid=(NB,),
        in_specs=[
            pl.BlockSpec((BLK, D), lambda i: (i, 0)),
            pl.BlockSpec((BLK, D), lambda i: (i, 0)),
        ],
        out_specs=pl.BlockSpec((BLK, D), lambda i: (i, 0)),
        out_shape=jax.ShapeDtypeStruct((B, D), jnp.float32),
    )(batch_pred, batch_truth)

    parts, cnts = _sc_segsum_call(offsets, batch_label)

    mean_p, w_p = pl.pallas_call(
        _weights_body,
        out_shape=[
            jax.ShapeDtypeStruct((C, D), jnp.float32),
            jax.ShapeDtypeStruct((C, D), jnp.float32),
        ],
    )(parts, cnts)
    return (mean_p, w_p)
